# Initial kernel scaffold; baseline (speedup 1.0000x reference)
import functools

import jax
import jax.numpy as jnp
from jax import lax
from jax.experimental import pallas as pl
from jax.experimental.pallas import tpu as pltpu
from jax.experimental.pallas import tpu_sc as plsc

GAZ_SIZE = 1000000
EMBED_DIM = 32
B = 4096
L = 200
N = B * L

_INFO = plsc.get_sparse_core_info()
NC = _INFO.num_cores
NS = _INFO.num_subcores
NW = NC * NS
PER_W = N // NW
CHUNK = 1600
NCHUNKS = PER_W // CHUNK


def _body(table_hbm, idx_hbm, out_hbm, idx_v, rows_v, gsem):
    wid = lax.axis_index("s") * NC + lax.axis_index("c")
    base = wid * PER_W
    for g in range(NCHUNKS):
        off = base + g * CHUNK
        pltpu.sync_copy(idx_hbm.at[pl.ds(off, CHUNK)], idx_v)
        pltpu.async_copy(table_hbm.at[idx_v], rows_v, gsem).wait()
        pltpu.sync_copy(rows_v, out_hbm.at[pl.ds(off, CHUNK)])


@jax.jit
def kernel(gazetteer_ids, weight):
    flat_ids = gazetteer_ids.reshape(N).astype(jnp.int32)
    mesh = plsc.VectorSubcoreMesh(core_axis_name="c", subcore_axis_name="s")
    out = pl.kernel(
        _body,
        out_type=jax.ShapeDtypeStruct((N, EMBED_DIM), jnp.float32),
        mesh=mesh,
        scratch_types=[
            pltpu.VMEM((CHUNK,), jnp.int32),
            pltpu.VMEM((CHUNK, EMBED_DIM), jnp.float32),
            pltpu.SemaphoreType.DMA,
        ],
    )(weight, flat_ids)
    return out.reshape(B, L, EMBED_DIM)


# speedup vs baseline: 1.4784x; 1.4784x over previous
import functools

import jax
import jax.numpy as jnp
from jax import lax
from jax.experimental import pallas as pl
from jax.experimental.pallas import tpu as pltpu
from jax.experimental.pallas import tpu_sc as plsc

GAZ_SIZE = 1000000
EMBED_DIM = 32
B = 4096
L = 200
N = B * L

_INFO = plsc.get_sparse_core_info()
NC = _INFO.num_cores
NS = _INFO.num_subcores
NW = NC * NS
PER_W = N // NW
CHUNK = 1600
NCHUNKS = PER_W // CHUNK


def _body(table_hbm, idx_hbm, out_hbm, idx_v, rows_v, gsem):
    wid = lax.axis_index("s") * NC + lax.axis_index("c")
    base = wid * PER_W
    for g in range(NCHUNKS):
        off = base + g * CHUNK
        pltpu.sync_copy(idx_hbm.at[pl.ds(off, CHUNK)], idx_v)
        pltpu.async_copy(table_hbm.at[idx_v], rows_v, gsem).wait()
        pltpu.sync_copy(rows_v, out_hbm.at[pl.ds(off, CHUNK)])


@jax.jit
def kernel(gazetteer_ids, weight):
    flat_ids = gazetteer_ids.reshape(N).astype(jnp.int32)
    mesh = plsc.VectorSubcoreMesh(core_axis_name="c", subcore_axis_name="s")
    out = pl.kernel(
        _body,
        out_type=jax.ShapeDtypeStruct((N, EMBED_DIM), jnp.float32),
        mesh=mesh,
        scratch_types=[
            pltpu.VMEM((CHUNK,), jnp.int32),
            pltpu.VMEM((CHUNK, EMBED_DIM), jnp.float32),
            pltpu.SemaphoreType.DMA,
        ],
        compiler_params=pltpu.CompilerParams(use_tc_tiling_on_sc=False),
    )(weight, flat_ids)
    return out.reshape(B, L, EMBED_DIM)


# speedup vs baseline: 1.4910x; 1.0085x over previous
import functools

import jax
import jax.numpy as jnp
from jax import lax
from jax.experimental import pallas as pl
from jax.experimental.pallas import tpu as pltpu
from jax.experimental.pallas import tpu_sc as plsc

GAZ_SIZE = 1000000
EMBED_DIM = 32
B = 4096
L = 200
N = B * L

_INFO = plsc.get_sparse_core_info()
NC = _INFO.num_cores
NS = _INFO.num_subcores
NW = NC * NS
PER_W = N // NW
CHUNK = 1600
NCHUNKS = PER_W // CHUNK


def _body(table_hbm, idx_hbm, out_hbm,
          idx0, idx1, rows0, rows1, gsem0, gsem1, osem0, osem1):
    wid = lax.axis_index("s") * NC + lax.axis_index("c")
    base = wid * PER_W
    idx_v = (idx0, idx1)
    rows_v = (rows0, rows1)
    gsem = (gsem0, gsem1)
    osem = (osem0, osem1)

    hg = [None] * NCHUNKS
    ho = [None] * NCHUNKS
    pltpu.sync_copy(idx_hbm.at[pl.ds(base, CHUNK)], idx_v[0])
    hg[0] = pltpu.async_copy(table_hbm.at[idx_v[0]], rows_v[0], gsem[0])
    for g in range(NCHUNKS):
        b = g % 2
        hg[g].wait()
        ho[g] = pltpu.async_copy(
            rows_v[b], out_hbm.at[pl.ds(base + g * CHUNK, CHUNK)], osem[b])
        if g + 1 < NCHUNKS:
            pltpu.sync_copy(
                idx_hbm.at[pl.ds(base + (g + 1) * CHUNK, CHUNK)], idx_v[b])
            if g >= 1:
                ho[g - 1].wait()
            hg[g + 1] = pltpu.async_copy(
                table_hbm.at[idx_v[b]], rows_v[1 - b], gsem[1 - b])
    ho[NCHUNKS - 1].wait()


@jax.jit
def kernel(gazetteer_ids, weight):
    flat_ids = gazetteer_ids.reshape(N).astype(jnp.int32)
    mesh = plsc.VectorSubcoreMesh(core_axis_name="c", subcore_axis_name="s")
    out = pl.kernel(
        _body,
        out_type=jax.ShapeDtypeStruct((N, EMBED_DIM), jnp.float32),
        mesh=mesh,
        scratch_types=[
            pltpu.VMEM((CHUNK,), jnp.int32),
            pltpu.VMEM((CHUNK,), jnp.int32),
            pltpu.VMEM((CHUNK, EMBED_DIM), jnp.float32),
            pltpu.VMEM((CHUNK, EMBED_DIM), jnp.float32),
            pltpu.SemaphoreType.DMA,
            pltpu.SemaphoreType.DMA,
            pltpu.SemaphoreType.DMA,
            pltpu.SemaphoreType.DMA,
        ],
        compiler_params=pltpu.CompilerParams(use_tc_tiling_on_sc=False),
    )(weight, flat_ids)
    return out.reshape(B, L, EMBED_DIM)
